# baseline (speedup 1.0000x reference)
import functools

import jax
import jax.numpy as jnp
from jax import lax
from jax.experimental import pallas as pl
from jax.experimental.pallas import tpu as pltpu
from jax.experimental.pallas import tpu_sc as plsc

NC, NS, LANES = 2, 16, 16
NW = NC * NS
MASKHI = jnp.int32(-65536)


def _sc_gather_sum(B, L, V, D, VP):
    CB = 64
    n_chunks = B // CB
    n_groups = CB // LANES
    LP = L + 1
    CHUNK = CB * LP
    PIECES = VP // 128

    mesh = plsc.VectorSubcoreMesh(
        core_axis_name="c", subcore_axis_name="s", num_cores=NC, num_subcores=NS
    )

    @functools.partial(
        pl.kernel,
        out_type=jax.ShapeDtypeStruct((D, B), jnp.float32),
        mesh=mesh,
        compiler_params=pltpu.CompilerParams(
            use_tc_tiling_on_sc=False, needs_layout_passes=False,
            disable_bounds_checks=True
        ),
        scratch_types=[
            pltpu.VMEM((PIECES, 128), jnp.int32),
            pltpu.VMEM((2, CHUNK), jnp.int32),
            pltpu.VMEM((2, B), jnp.float32),
            pltpu.VMEM((2, LANES), jnp.float32),
            pltpu.SemaphoreType.DMA,
            pltpu.SemaphoreType.DMA,
            pltpu.SemaphoreType.DMA,
        ],
    )
    def k(tok_hbm, wp_hbm, bb_hbm, out_hbm, wrow_v, tok_v, orow_v, bias_v,
          sem0, sem1, semw):
        cid = lax.axis_index("c")
        sid = lax.axis_index("s")
        wid = sid * NC + cid
        sems = (sem0, sem1)
        base = lax.iota(jnp.int32, 16) * LP
        boffs = [base + g * LANES * LP for g in range(n_groups)]

        wcp = pltpu.async_copy(wp_hbm.at[wid // 8, :, wid % 8, :], wrow_v,
                               semw)
        pending = pltpu.async_copy(
            tok_hbm.at[pl.ds(0, CHUNK)], tok_v.at[0], sems[0])
        pltpu.sync_copy(bb_hbm.at[wid], bias_v.at[0])
        pltpu.sync_copy(bb_hbm.at[wid + NW], bias_v.at[1])
        bias_hi = bias_v[0, :]
        bias_lo = bias_v[1, :]
        wcp.wait()
        for c in range(n_chunks):
            buf = c % 2
            nxt = None
            if c + 1 < n_chunks:
                nxt = pltpu.async_copy(
                    tok_hbm.at[pl.ds((c + 1) * CHUNK, CHUNK)],
                    tok_v.at[(c + 1) % 2], sems[(c + 1) % 2])
            pending.wait()

            def lbody(l, accs, _buf=buf):
                his, los = accs
                nh, nl = [], []
                for g in range(n_groups):
                    ti = plsc.load_gather(tok_v.at[_buf], [boffs[g] + l])
                    x = plsc.load_gather(
                        wrow_v, [lax.shift_right_logical(ti, 7), ti & 127])
                    hi = plsc.bitcast(x & MASKHI, jnp.float32)
                    lo = plsc.bitcast(lax.shift_left(x, 16), jnp.float32)
                    nh.append(his[g] + hi)
                    nl.append(los[g] + lo)
                return tuple(nh), tuple(nl)

            accs = lax.fori_loop(
                0, L, lbody,
                ((bias_hi,) * n_groups, (bias_lo,) * n_groups), unroll=2)
            for g in range(n_groups):
                orow_v[0, pl.ds(c * CB + g * LANES, LANES)] = accs[0][g]
                orow_v[1, pl.ds(c * CB + g * LANES, LANES)] = accs[1][g]
            pending = nxt
        pltpu.sync_copy(orow_v.at[0], out_hbm.at[wid])
        pltpu.sync_copy(orow_v.at[1], out_hbm.at[wid + NW])

    return k


def kernel(tokens, W, b):
    B, L = tokens.shape
    D, V = W.shape
    VP = ((V + 127) // 128) * 128
    tok_pad = jnp.pad(tokens.astype(jnp.int32), ((0, 0), (0, 1)))
    tok_flat = tok_pad.reshape(B * (L + 1))
    Wu = lax.bitcast_convert_type(
        jnp.pad(W, ((0, 0), (0, VP - V))), jnp.uint32)
    Wr = Wu + jnp.uint32(0x7FFF) + ((Wu >> 16) & jnp.uint32(1))
    Wpack = lax.bitcast_convert_type(
        (Wr[: D // 2] & jnp.uint32(0xFFFF0000)) | (Wr[D // 2:] >> 16),
        jnp.int32)
    Wp4 = Wpack.reshape(4, 8, VP // 128, 128).transpose(0, 2, 1, 3)
    bb = jnp.broadcast_to(b[:, None], (D, LANES))
    outT = _sc_gather_sum(B, L, V, D, VP)(tok_flat, Wp4, bb)
    return outT.T[:, None, :]


# speedup vs baseline: 1.0007x; 1.0007x over previous
import functools

import jax
import jax.numpy as jnp
from jax import lax
from jax.experimental import pallas as pl
from jax.experimental.pallas import tpu as pltpu
from jax.experimental.pallas import tpu_sc as plsc

NC, NS, LANES = 2, 16, 16
NW = NC * NS
MASKHI = jnp.int32(-65536)


def _sc_gather_sum(B, L, V, D, VP):
    CB = 64
    n_chunks = B // CB
    n_groups = CB // LANES
    LP = L + 1
    CHUNK = CB * LP
    PIECES = VP // 128

    mesh = plsc.VectorSubcoreMesh(
        core_axis_name="c", subcore_axis_name="s", num_cores=NC, num_subcores=NS
    )

    @functools.partial(
        pl.kernel,
        out_type=jax.ShapeDtypeStruct((D, B), jnp.float32),
        mesh=mesh,
        compiler_params=pltpu.CompilerParams(
            use_tc_tiling_on_sc=False, needs_layout_passes=False,
            disable_bounds_checks=True
        ),
        scratch_types=[
            pltpu.VMEM((PIECES, 128), jnp.int32),
            pltpu.VMEM((2, CHUNK), jnp.int32),
            pltpu.VMEM((2, B), jnp.float32),
            pltpu.VMEM((2, LANES), jnp.float32),
            pltpu.SemaphoreType.DMA,
            pltpu.SemaphoreType.DMA,
            pltpu.SemaphoreType.DMA,
        ],
    )
    def k(tok_hbm, wp_hbm, bb_hbm, out_hbm, wrow_v, tok_v, orow_v, bias_v,
          sem0, sem1, semw):
        cid = lax.axis_index("c")
        sid = lax.axis_index("s")
        wid = sid * NC + cid
        sems = (sem0, sem1)
        base = lax.iota(jnp.int32, 16) * LP
        boffs = [base + g * LANES * LP for g in range(n_groups)]

        wcp = pltpu.async_copy(wp_hbm.at[wid // 8, :, wid % 8, :], wrow_v,
                               semw)
        pending = pltpu.async_copy(
            tok_hbm.at[pl.ds(0, CHUNK)], tok_v.at[0], sems[0])
        pltpu.sync_copy(bb_hbm.at[wid], bias_v.at[0])
        pltpu.sync_copy(bb_hbm.at[wid + NW], bias_v.at[1])
        bias_hi = bias_v[0, :]
        bias_lo = bias_v[1, :]
        wcp.wait()
        for c in range(n_chunks):
            buf = c % 2
            nxt = None
            if c + 1 < n_chunks:
                nxt = pltpu.async_copy(
                    tok_hbm.at[pl.ds((c + 1) * CHUNK, CHUNK)],
                    tok_v.at[(c + 1) % 2], sems[(c + 1) % 2])
            pending.wait()

            def lbody(l, accs, _buf=buf):
                his, los = accs
                nh, nl = [], []
                for g in range(n_groups):
                    ti = plsc.load_gather(tok_v.at[_buf], [boffs[g] + l])
                    x = plsc.load_gather(
                        wrow_v, [lax.shift_right_logical(ti, 7), ti & 127])
                    hi = plsc.bitcast(x & MASKHI, jnp.float32)
                    lo = plsc.bitcast(lax.shift_left(x, 16), jnp.float32)
                    nh.append(his[g] + hi)
                    nl.append(los[g] + lo)
                return tuple(nh), tuple(nl)

            accs = lax.fori_loop(
                0, L, lbody,
                ((bias_hi,) * n_groups, (bias_lo,) * n_groups), unroll=2)
            for g in range(n_groups):
                orow_v[0, pl.ds(c * CB + g * LANES, LANES)] = accs[0][g]
                orow_v[1, pl.ds(c * CB + g * LANES, LANES)] = accs[1][g]
            pending = nxt
        pltpu.sync_copy(orow_v.at[0], out_hbm.at[wid])
        pltpu.sync_copy(orow_v.at[1], out_hbm.at[wid + NW])

    return k


def kernel(tokens, W, b):
    B, L = tokens.shape
    D, V = W.shape
    VP = ((V + 127) // 128) * 128
    tok_pad = jnp.pad(tokens.astype(jnp.int32), ((0, 0), (0, 1)))
    tok_flat = tok_pad.reshape(B * (L + 1))
    Wu = lax.bitcast_convert_type(
        jnp.pad(W, ((0, 0), (0, VP - V))), jnp.uint32)
    Wt = Wu.reshape(8, 8, VP // 128, 128).transpose(0, 2, 1, 3)
    Wr = Wt + jnp.uint32(0x7FFF) + ((Wt >> 16) & jnp.uint32(1))
    Wp4 = lax.bitcast_convert_type(
        (Wr[:4] & jnp.uint32(0xFFFF0000)) | (Wr[4:] >> 16),
        jnp.int32)
    bb = jnp.broadcast_to(b[:, None], (D, LANES))
    outT = _sc_gather_sum(B, L, V, D, VP)(tok_flat, Wp4, bb)
    return outT.T[:, None, :]


# speedup vs baseline: 1.0192x; 1.0185x over previous
import functools

import jax
import jax.numpy as jnp
from jax import lax
from jax.experimental import pallas as pl
from jax.experimental.pallas import tpu as pltpu
from jax.experimental.pallas import tpu_sc as plsc

NC, NS, LANES = 2, 16, 16
NW = NC * NS
MASKHI = -65536


def _sc_gather_sum(B, L, V, D, VP):
    CB = 64
    n_chunks = B // CB
    n_groups = CB // LANES
    LP = L + 1
    CHUNK = CB * LP
    PIECES = VP // 128

    mesh = plsc.VectorSubcoreMesh(
        core_axis_name="c", subcore_axis_name="s", num_cores=NC, num_subcores=NS
    )

    @functools.partial(
        pl.kernel,
        out_type=jax.ShapeDtypeStruct((D, B), jnp.float32),
        mesh=mesh,
        compiler_params=pltpu.CompilerParams(
            use_tc_tiling_on_sc=False, needs_layout_passes=False,
            disable_bounds_checks=True
        ),
        scratch_types=[
            pltpu.VMEM((PIECES, 128), jnp.int32),
            pltpu.VMEM((2, CHUNK), jnp.int32),
            pltpu.VMEM((2, B), jnp.float32),
            pltpu.VMEM((2, LANES), jnp.float32),
            pltpu.SemaphoreType.DMA,
            pltpu.SemaphoreType.DMA,
            pltpu.SemaphoreType.DMA,
        ],
    )
    def k(tok_hbm, wp_hbm, bb_hbm, out_hbm, wrow_v, tok_v, orow_v, bias_v,
          sem0, sem1, semw):
        cid = lax.axis_index("c")
        sid = lax.axis_index("s")
        wid = sid * NC + cid
        sems = (sem0, sem1)
        base = lax.iota(jnp.int32, 16) * LP
        boffs = [base + g * LANES * LP for g in range(n_groups)]

        wcp = pltpu.async_copy(wp_hbm.at[wid // 8, :, wid % 8, :], wrow_v,
                               semw)
        pltpu.async_copy(tok_hbm.at[pl.ds(0, CHUNK)], tok_v.at[0], sems[0])
        pltpu.async_copy(tok_hbm.at[pl.ds(CHUNK, CHUNK)], tok_v.at[1],
                         sems[1])
        pltpu.sync_copy(bb_hbm.at[wid], bias_v.at[0])
        pltpu.sync_copy(bb_hbm.at[wid + NW], bias_v.at[1])
        bias_hi = bias_v[0, :]
        bias_lo = bias_v[1, :]
        wcp.wait()

        def process_chunk(c, buf):
            pltpu.make_async_copy(
                tok_hbm.at[pl.ds(c * CHUNK, CHUNK)], tok_v.at[buf],
                sems[buf]).wait()

            def lbody(l, accs):
                his, los = accs
                nh, nl = [], []
                for g in range(n_groups):
                    ti = plsc.load_gather(tok_v.at[buf], [boffs[g] + l])
                    x = plsc.load_gather(
                        wrow_v, [lax.shift_right_logical(ti, 7), ti & 127])
                    hi = plsc.bitcast(x & jnp.int32(MASKHI), jnp.float32)
                    lo = plsc.bitcast(lax.shift_left(x, 16), jnp.float32)
                    nh.append(his[g] + hi)
                    nl.append(los[g] + lo)
                return tuple(nh), tuple(nl)

            accs = lax.fori_loop(
                0, L, lbody,
                ((bias_hi,) * n_groups, (bias_lo,) * n_groups), unroll=2)
            for g in range(n_groups):
                orow_v[0, pl.ds(c * CB + g * LANES, LANES)] = accs[0][g]
                orow_v[1, pl.ds(c * CB + g * LANES, LANES)] = accs[1][g]

        def jbody(j, carry):
            c0 = 2 * j
            process_chunk(c0, 0)
            pltpu.async_copy(
                tok_hbm.at[pl.ds((c0 + 2) * CHUNK, CHUNK)], tok_v.at[0],
                sems[0])
            process_chunk(c0 + 1, 1)
            pltpu.async_copy(
                tok_hbm.at[pl.ds((c0 + 3) * CHUNK, CHUNK)], tok_v.at[1],
                sems[1])
            return carry

        lax.fori_loop(0, n_chunks // 2 - 1, jbody, jnp.int32(0))
        process_chunk(n_chunks - 2, 0)
        process_chunk(n_chunks - 1, 1)
        pltpu.sync_copy(orow_v.at[0], out_hbm.at[wid])
        pltpu.sync_copy(orow_v.at[1], out_hbm.at[wid + NW])

    return k


def kernel(tokens, W, b):
    B, L = tokens.shape
    D, V = W.shape
    VP = ((V + 127) // 128) * 128
    tok_pad = jnp.pad(tokens.astype(jnp.int32), ((0, 0), (0, 1)))
    tok_flat = tok_pad.reshape(B * (L + 1))
    Wu = lax.bitcast_convert_type(
        jnp.pad(W, ((0, 0), (0, VP - V))), jnp.uint32)
    Wt = Wu.reshape(8, 8, VP // 128, 128).transpose(0, 2, 1, 3)
    Wr = Wt + jnp.uint32(0x7FFF) + ((Wt >> 16) & jnp.uint32(1))
    Wp4 = lax.bitcast_convert_type(
        (Wr[:4] & jnp.uint32(0xFFFF0000)) | (Wr[4:] >> 16),
        jnp.int32)
    bb = jnp.broadcast_to(b[:, None], (D, LANES))
    outT = _sc_gather_sum(B, L, V, D, VP)(tok_flat, Wp4, bb)
    return outT.T[:, None, :]
